# baseline (speedup 1.0000x reference)
import functools

import jax
import jax.numpy as jnp
from jax import lax
from jax.experimental import pallas as pl
from jax.experimental.pallas import tpu as pltpu
from jax.experimental.pallas import tpu_sc as plsc

VOCAB = 1000000
EMB_DIM = 64
BATCH = 16384
HIST = 50

NC = 2
NS = 16
NW = NC * NS

TOTAL = BATCH * HIST
B_PER_W = BATCH // NW
PER_W = B_PER_W * HIST
CHUNK = 256
QN = CHUNK // 128
NCHUNK = PER_W // CHUNK
LANES = 16
ROW_BYTES_OUT = BATCH * 4
TILE_ROW = 128 * 1024

_mesh = plsc.VectorSubcoreMesh(
    core_axis_name="c", subcore_axis_name="s", num_cores=NC, num_subcores=NS
)


@functools.partial(
    pl.kernel,
    out_type=jax.ShapeDtypeStruct((HIST * 8 * TILE_ROW,), jnp.float32),
    mesh=_mesh,
    scratch_types=[
        pltpu.VMEM((PER_W,), jnp.int32),
        [pltpu.VMEM((CHUNK, EMB_DIM), jnp.float32) for _ in range(2)],
        [pltpu.VMEM((8 * QN * 1024,), jnp.float32) for _ in range(2)],
        pltpu.SemaphoreType.DMA,
        [pltpu.SemaphoreType.DMA for _ in range(2)],
        [pltpu.SemaphoreType.DMA for _ in range(2)],
    ],
    compiler_params=pltpu.CompilerParams(
        use_tc_tiling_on_sc=False, needs_layout_passes=False
    ),
)
def _emb_lookup(idx_hbm, table_hbm, out_hbm, idx_all, rows, rowsT, isem, gsems, wsems):
    wid = lax.axis_index("s") * NC + lax.axis_index("c")
    b0 = wid * B_PER_W

    for h in range(HIST):
        pltpu.async_copy(
            idx_hbm.at[pl.ds(h * BATCH + b0, B_PER_W)],
            idx_all.at[pl.ds(h * B_PER_W, B_PER_W)],
            isem,
        )
    for h in range(HIST):
        pltpu.make_async_copy(
            idx_hbm.at[pl.ds(0, B_PER_W)], idx_all.at[pl.ds(0, B_PER_W)], isem
        ).wait()

    lane = lax.iota(jnp.int32, LANES)
    jvecs = [lane + jnp.int32(jb * LANES) for jb in range(CHUNK // LANES)]

    def fire_gather(c, b):
        pltpu.async_copy(
            table_hbm.at[idx_all.at[pl.ds(c * CHUNK, CHUNK)]], rows[b], gsems[b]
        )

    def wait_gather(b):
        pltpu.make_async_copy(
            table_hbm.at[idx_all.at[pl.ds(0, CHUNK)]], rows[b], gsems[b]
        ).wait()

    def fire_writeback(c, b):
        h = c // 2
        tc0 = 4 * wid + (c % 2) * QN
        for tr in range(8):
            pltpu.async_copy(
                rowsT[b].at[pl.ds(tr * (QN * 1024), QN * 1024)],
                out_hbm.at[pl.ds((h * 8 + tr) * TILE_ROW + tc0 * 1024, QN * 1024)],
                wsems[b],
            )

    def wait_writeback(b):
        for tr in range(8):
            pltpu.make_async_copy(
                rowsT[b].at[pl.ds(0, QN * 1024)],
                out_hbm.at[pl.ds(0, QN * 1024)],
                wsems[b],
            ).wait()

    def transpose_chunk(b):
        def tbody(d, carry):
            tr = d >> 3
            s = d & 7
            dsplat = jnp.full((LANES,), d, dtype=jnp.int32)
            for jb in range(CHUNK // LANES):
                q, lb = divmod(jb * LANES, 128)
                off = tr * (QN * 1024) + q * 1024 + s * 128 + lb
                vals = plsc.load_gather(rows[b], [jvecs[jb], dsplat])
                rowsT[b][pl.ds(off, LANES)] = vals
            return carry

        lax.fori_loop(0, EMB_DIM, tbody, 0)

    fire_gather(0, 0)

    def body(g, carry):
        for b in range(2):
            c = g * 2 + b

            @pl.when(c + 1 < NCHUNK)
            def _():
                fire_gather(c + 1, 1 - b)

            wait_gather(b)

            @pl.when(g > 0)
            def _():
                wait_writeback(b)

            transpose_chunk(b)
            fire_writeback(c, b)
        return carry

    lax.fori_loop(0, NCHUNK // 2, body, 0)
    for b in range(2):
        wait_writeback(b)


def kernel(idx, emb):
    flat = jnp.transpose(idx).reshape(TOTAL)
    o3 = _emb_lookup(flat, emb)
    return (
        o3.reshape(HIST, 8, 128, 8, 128)
        .transpose(2, 4, 0, 1, 3)
        .reshape(BATCH, HIST, EMB_DIM)
    )


# speedup vs baseline: 1.7869x; 1.7869x over previous
import functools

import jax
import jax.numpy as jnp
from jax import lax
from jax.experimental import pallas as pl
from jax.experimental.pallas import tpu as pltpu
from jax.experimental.pallas import tpu_sc as plsc

VOCAB = 1000000
EMB_DIM = 64
BATCH = 16384
HIST = 50

NC = 2
NS = 16
NW = NC * NS

TOTAL = BATCH * HIST
B_PER_W = BATCH // NW
PER_W = B_PER_W * HIST
CHUNK = 256
QN = CHUNK // 128
NCHUNK = PER_W // CHUNK
LANES = 16
ROW_BYTES_OUT = BATCH * 4
TILE_ROW = 128 * 1024

_mesh = plsc.VectorSubcoreMesh(
    core_axis_name="c", subcore_axis_name="s", num_cores=NC, num_subcores=NS
)


@functools.partial(
    pl.kernel,
    out_type=jax.ShapeDtypeStruct((HIST * 8 * TILE_ROW,), jnp.float32),
    mesh=_mesh,
    scratch_types=[
        pltpu.VMEM((PER_W,), jnp.int32),
        [pltpu.VMEM((CHUNK, EMB_DIM), jnp.float32) for _ in range(2)],
        [pltpu.VMEM((8 * QN * 1024,), jnp.float32) for _ in range(2)],
        pltpu.SemaphoreType.DMA,
        [pltpu.SemaphoreType.DMA for _ in range(2)],
        [pltpu.SemaphoreType.DMA for _ in range(2)],
    ],
    compiler_params=pltpu.CompilerParams(
        use_tc_tiling_on_sc=False, needs_layout_passes=False
    ),
)
def _emb_lookup(idx_hbm, table_hbm, out_hbm, idx_all, rows, rowsT, isem, gsems, wsems):
    wid = lax.axis_index("s") * NC + lax.axis_index("c")
    b0 = wid * B_PER_W

    for h in range(HIST):
        pltpu.async_copy(
            idx_hbm.at[pl.ds(h * BATCH + b0, B_PER_W)],
            idx_all.at[pl.ds(h * B_PER_W, B_PER_W)],
            isem,
        )
    for h in range(HIST):
        pltpu.make_async_copy(
            idx_hbm.at[pl.ds(0, B_PER_W)], idx_all.at[pl.ds(0, B_PER_W)], isem
        ).wait()

    lane = lax.iota(jnp.int32, LANES)
    jvecs = [lane + jnp.int32(jb * LANES) for jb in range(CHUNK // LANES)]
    wstat = [
        lane + jnp.int32(((jb * LANES) // 128) * 1024 + (jb * LANES) % 128)
        for jb in range(CHUNK // LANES)
    ]

    def fire_gather(c, b):
        pltpu.async_copy(
            table_hbm.at[idx_all.at[pl.ds(c * CHUNK, CHUNK)]], rows[b], gsems[b]
        )

    def wait_gather(b):
        pltpu.make_async_copy(
            table_hbm.at[idx_all.at[pl.ds(0, CHUNK)]], rows[b], gsems[b]
        ).wait()

    def fire_writeback(c, b):
        h = c // 2
        tc0 = 4 * wid + (c % 2) * QN
        for tr in range(8):
            pltpu.async_copy(
                rowsT[b].at[pl.ds(tr * (QN * 1024), QN * 1024)],
                out_hbm.at[pl.ds((h * 8 + tr) * TILE_ROW + tc0 * 1024, QN * 1024)],
                wsems[b],
            )

    def wait_writeback(b):
        for tr in range(8):
            pltpu.make_async_copy(
                rowsT[b].at[pl.ds(0, QN * 1024)],
                out_hbm.at[pl.ds(0, QN * 1024)],
                wsems[b],
            ).wait()

    def transpose_chunk(b):
        def tbody(d0, carry):
            dmod = (jnp.full((LANES,), d0, dtype=jnp.int32) + lane) & 63
            wvec = (dmod >> 3) * jnp.int32(QN * 1024) + (dmod & 7) * jnp.int32(128)
            for jb in range(CHUNK // LANES):
                vals = plsc.load_gather(rows[b], [jvecs[jb], dmod])
                plsc.store_scatter(rowsT[b], [wvec + wstat[jb]], vals)
            return carry

        lax.fori_loop(0, EMB_DIM, tbody, 0)

    fire_gather(0, 0)

    def body(g, carry):
        for b in range(2):
            c = g * 2 + b

            @pl.when(c + 1 < NCHUNK)
            def _():
                fire_gather(c + 1, 1 - b)

            wait_gather(b)

            @pl.when(g > 0)
            def _():
                wait_writeback(b)

            transpose_chunk(b)
            fire_writeback(c, b)
        return carry

    lax.fori_loop(0, NCHUNK // 2, body, 0)
    for b in range(2):
        wait_writeback(b)


def kernel(idx, emb):
    flat = jnp.transpose(idx).reshape(TOTAL)
    o3 = _emb_lookup(flat, emb)
    return (
        o3.reshape(HIST, 8, 128, 8, 128)
        .transpose(2, 4, 0, 1, 3)
        .reshape(BATCH, HIST, EMB_DIM)
    )


# speedup vs baseline: 2.4230x; 1.3560x over previous
import functools

import jax
import jax.numpy as jnp
from jax import lax
from jax.experimental import pallas as pl
from jax.experimental.pallas import tpu as pltpu
from jax.experimental.pallas import tpu_sc as plsc

VOCAB = 1000000
EMB_DIM = 64
BATCH = 16384
HIST = 50

NC = 2
NS = 16
NW = NC * NS

TOTAL = BATCH * HIST
B_PER_W = BATCH // NW
PER_W = B_PER_W * HIST
CHUNK = 256
QN = CHUNK // 128
NCHUNK = PER_W // CHUNK
LANES = 16
ROW_BYTES_OUT = BATCH * 4
TILE_ROW = 128 * 1024

_mesh = plsc.VectorSubcoreMesh(
    core_axis_name="c", subcore_axis_name="s", num_cores=NC, num_subcores=NS
)


@functools.partial(
    pl.kernel,
    out_type=jax.ShapeDtypeStruct((HIST * 8 * TILE_ROW,), jnp.float32),
    mesh=_mesh,
    scratch_types=[
        pltpu.VMEM((PER_W,), jnp.int32),
        [pltpu.VMEM((CHUNK, EMB_DIM), jnp.float32) for _ in range(2)],
        [pltpu.VMEM((8 * QN * 1024,), jnp.float32) for _ in range(2)],
        pltpu.SemaphoreType.DMA,
        [pltpu.SemaphoreType.DMA for _ in range(2)],
        [pltpu.SemaphoreType.DMA for _ in range(2)],
    ],
    compiler_params=pltpu.CompilerParams(
        use_tc_tiling_on_sc=False, needs_layout_passes=False
    ),
)
def _emb_lookup(idx_hbm, table_hbm, out_hbm, idx_all, rows, rowsT, isem, gsems, wsems):
    wid = lax.axis_index("s") * NC + lax.axis_index("c")
    b0 = wid * B_PER_W

    for h in range(HIST):
        pltpu.async_copy(
            idx_hbm.at[pl.ds(h * BATCH + b0, B_PER_W)],
            idx_all.at[pl.ds(h * B_PER_W, B_PER_W)],
            isem,
        )
    for h in range(HIST):
        pltpu.make_async_copy(
            idx_hbm.at[pl.ds(0, B_PER_W)], idx_all.at[pl.ds(0, B_PER_W)], isem
        ).wait()

    lane = lax.iota(jnp.int32, LANES)
    jvecs = [lane + jnp.int32(jb * LANES) for jb in range(CHUNK // LANES)]
    wstat = [
        lane + jnp.int32(((jb * LANES) // 128) * 1024 + (jb * LANES) % 128)
        for jb in range(CHUNK // LANES)
    ]

    def fire_gather(c, b):
        pltpu.async_copy(
            table_hbm.at[idx_all.at[pl.ds(c * CHUNK, CHUNK)]], rows[b], gsems[b]
        )

    def wait_gather(b):
        pltpu.make_async_copy(
            table_hbm.at[idx_all.at[pl.ds(0, CHUNK)]], rows[b], gsems[b]
        ).wait()

    def fire_writeback(c, b):
        h = c // 2
        tc0 = 4 * wid + (c % 2) * QN
        for tr in range(8):
            pltpu.async_copy(
                rowsT[b].at[pl.ds(tr * (QN * 1024), QN * 1024)],
                out_hbm.at[pl.ds((h * 8 + tr) * TILE_ROW + tc0 * 1024, QN * 1024)],
                wsems[b],
            )

    def wait_writeback(b):
        for tr in range(8):
            pltpu.make_async_copy(
                rowsT[b].at[pl.ds(0, QN * 1024)],
                out_hbm.at[pl.ds(0, QN * 1024)],
                wsems[b],
            ).wait()

    def transpose_chunk(b):
        @plsc.parallel_loop(0, EMB_DIM, 1, unroll=2)
        def tbody(d0):
            dmod = (jnp.full((LANES,), d0, dtype=jnp.int32) + lane) & 63
            wvec = (dmod >> 3) * jnp.int32(QN * 1024) + (dmod & 7) * jnp.int32(128)
            for jb in range(CHUNK // LANES):
                vals = plsc.load_gather(rows[b], [jvecs[jb], dmod])
                plsc.store_scatter(rowsT[b], [wvec + wstat[jb]], vals)

    fire_gather(0, 0)

    def body(g, carry):
        for b in range(2):
            c = g * 2 + b

            @pl.when(c + 1 < NCHUNK)
            def _():
                fire_gather(c + 1, 1 - b)

            wait_gather(b)

            @pl.when(g > 0)
            def _():
                wait_writeback(b)

            transpose_chunk(b)
            fire_writeback(c, b)
        return carry

    lax.fori_loop(0, NCHUNK // 2, body, 0)
    for b in range(2):
        wait_writeback(b)


def kernel(idx, emb):
    flat = jnp.transpose(idx).reshape(TOTAL)
    o3 = _emb_lookup(flat, emb)
    return (
        o3.reshape(HIST, 8, 128, 8, 128)
        .transpose(2, 4, 0, 1, 3)
        .reshape(BATCH, HIST, EMB_DIM)
    )


# speedup vs baseline: 2.4245x; 1.0006x over previous
import functools

import jax
import jax.numpy as jnp
from jax import lax
from jax.experimental import pallas as pl
from jax.experimental.pallas import tpu as pltpu
from jax.experimental.pallas import tpu_sc as plsc

VOCAB = 1000000
EMB_DIM = 64
BATCH = 16384
HIST = 50

NC = 2
NS = 16
NW = NC * NS

TOTAL = BATCH * HIST
B_PER_W = BATCH // NW
PER_W = B_PER_W * HIST
CHUNK = 256
QN = CHUNK // 128
NCHUNK = PER_W // CHUNK
LANES = 16
ROW_BYTES_OUT = BATCH * 4
TILE_ROW = 128 * 1024

_mesh = plsc.VectorSubcoreMesh(
    core_axis_name="c", subcore_axis_name="s", num_cores=NC, num_subcores=NS
)


@functools.partial(
    pl.kernel,
    out_type=jax.ShapeDtypeStruct((HIST * 8 * TILE_ROW,), jnp.float32),
    mesh=_mesh,
    scratch_types=[
        pltpu.VMEM((PER_W,), jnp.int32),
        [pltpu.VMEM((CHUNK, EMB_DIM), jnp.float32) for _ in range(2)],
        [pltpu.VMEM((8 * QN * 1024,), jnp.float32) for _ in range(2)],
        pltpu.SemaphoreType.DMA,
        [pltpu.SemaphoreType.DMA for _ in range(2)],
        [pltpu.SemaphoreType.DMA for _ in range(2)],
    ],
    compiler_params=pltpu.CompilerParams(
        use_tc_tiling_on_sc=False, needs_layout_passes=False
    ),
)
def _emb_lookup(idx_hbm, table_hbm, out_hbm, idx_all, rows, rowsT, isem, gsems, wsems):
    wid = lax.axis_index("s") * NC + lax.axis_index("c")
    b0 = wid * B_PER_W

    for h in range(HIST):
        pltpu.async_copy(
            idx_hbm.at[pl.ds(h * BATCH + b0, B_PER_W)],
            idx_all.at[pl.ds(h * B_PER_W, B_PER_W)],
            isem,
        )
    for h in range(HIST):
        pltpu.make_async_copy(
            idx_hbm.at[pl.ds(0, B_PER_W)], idx_all.at[pl.ds(0, B_PER_W)], isem
        ).wait()

    lane = lax.iota(jnp.int32, LANES)
    jvecs = [lane + jnp.int32(jb * LANES) for jb in range(CHUNK // LANES)]
    wstat = [
        lane + jnp.int32(((jb * LANES) // 128) * 1024 + (jb * LANES) % 128)
        for jb in range(CHUNK // LANES)
    ]

    def fire_gather(c, b):
        pltpu.async_copy(
            table_hbm.at[idx_all.at[pl.ds(c * CHUNK, CHUNK)]], rows[b], gsems[b]
        )

    def wait_gather(b):
        pltpu.make_async_copy(
            table_hbm.at[idx_all.at[pl.ds(0, CHUNK)]], rows[b], gsems[b]
        ).wait()

    def fire_writeback(c, b):
        h = c // 2
        tc0 = 4 * wid + (c % 2) * QN
        for tr in range(8):
            pltpu.async_copy(
                rowsT[b].at[pl.ds(tr * (QN * 1024), QN * 1024)],
                out_hbm.at[pl.ds((h * 8 + tr) * TILE_ROW + tc0 * 1024, QN * 1024)],
                wsems[b],
            )

    def wait_writeback(b):
        for tr in range(8):
            pltpu.make_async_copy(
                rowsT[b].at[pl.ds(0, QN * 1024)],
                out_hbm.at[pl.ds(0, QN * 1024)],
                wsems[b],
            ).wait()

    def transpose_chunk(b):
        @plsc.parallel_loop(0, EMB_DIM, 1, unroll=4)
        def tbody(d0):
            dmod = (jnp.full((LANES,), d0, dtype=jnp.int32) + lane) & 63
            wvec = (dmod >> 3) * jnp.int32(QN * 1024) + (dmod & 7) * jnp.int32(128)
            for jb in range(CHUNK // LANES):
                vals = plsc.load_gather(rows[b], [jvecs[jb], dmod])
                plsc.store_scatter(rowsT[b], [wvec + wstat[jb]], vals)

    fire_gather(0, 0)

    def body(g, carry):
        for b in range(2):
            c = g * 2 + b

            @pl.when(c + 1 < NCHUNK)
            def _():
                fire_gather(c + 1, 1 - b)

            wait_gather(b)

            @pl.when(g > 0)
            def _():
                wait_writeback(b)

            transpose_chunk(b)
            fire_writeback(c, b)
        return carry

    lax.fori_loop(0, NCHUNK // 2, body, 0)
    for b in range(2):
        wait_writeback(b)


def kernel(idx, emb):
    flat = jnp.transpose(idx).reshape(TOTAL)
    o3 = _emb_lookup(flat, emb)
    return (
        o3.reshape(HIST, 8, 128, 8, 128)
        .transpose(2, 4, 0, 1, 3)
        .reshape(BATCH, HIST, EMB_DIM)
    )
